# baseline (speedup 1.0000x reference)
import functools

import jax
import jax.numpy as jnp
from jax.experimental import pallas as pl
from jax.experimental.pallas import tpu as pltpu

_DROP_PROB = 0.1
_MIN_KEEP = 1


def _select_kernel(imp_ref, noise_ref, keep_ref, scaled_ref, *, k, scale):
    rows, lanes = imp_ref.shape
    n = rows * lanes
    score = 1.0 / (imp_ref[...] + 1e-8) + noise_ref[...]
    bits = jax.lax.bitcast_convert_type(score, jnp.int32)

    def _bits_body(_, carry):
        lo, hi = carry
        mid = lo + (hi - lo) // 2
        cnt = jnp.sum((bits <= mid).astype(jnp.int32))
        ge = cnt >= k
        return jnp.where(ge, lo, mid + 1), jnp.where(ge, mid, hi)

    t, _ = jax.lax.fori_loop(
        0, 31, _bits_body, (jnp.int32(0), jnp.int32(2**31 - 1))
    )

    n_less = jnp.sum((bits < t).astype(jnp.int32))
    rem = k - n_less
    eq = bits == t
    idx = (
        jax.lax.broadcasted_iota(jnp.int32, (rows, lanes), 0) * lanes
        + jax.lax.broadcasted_iota(jnp.int32, (rows, lanes), 1)
    )

    def _idx_body(_, carry):
        lo, hi = carry
        mid = lo + (hi - lo) // 2
        cnt = jnp.sum((eq & (idx < mid)).astype(jnp.int32))
        ge = cnt >= rem
        return jnp.where(ge, lo, mid + 1), jnp.where(ge, mid, hi)

    m, _ = jax.lax.fori_loop(0, 16, _idx_body, (jnp.int32(0), jnp.int32(n)))

    keep = (bits < t) | (eq & (idx < m))
    keep_f = keep.astype(keep_ref.dtype)
    keep_ref[...] = keep_f
    scaled_ref[...] = keep_f * scale


def _apply_kernel(x_ref, m_ref, o_ref):
    o_ref[...] = x_ref[...] * m_ref[0:1, :][:, None, :]


def kernel(x, importance):
    b, n, c = x.shape
    num_keep = max(_MIN_KEEP, int(n * (1.0 - _DROP_PROB)))
    scale = n / num_keep
    noise = (
        jax.random.uniform(jax.random.key(42), importance.shape,
                           dtype=importance.dtype)
        * 0.5
    )
    lanes = 128
    rows = n // lanes
    keep2, scaled2 = pl.pallas_call(
        functools.partial(_select_kernel, k=num_keep, scale=scale),
        out_shape=(
            jax.ShapeDtypeStruct((rows, lanes), x.dtype),
            jax.ShapeDtypeStruct((rows, lanes), x.dtype),
        ),
    )(importance.reshape(rows, lanes), noise.reshape(rows, lanes))

    keep_mask = keep2.reshape(n)
    xt = jnp.transpose(x, (0, 2, 1))
    mask_row = jnp.broadcast_to(scaled2.reshape(n)[None, :], (8, n))

    w = n // 2
    out_t = pl.pallas_call(
        _apply_kernel,
        grid=(n // w, b // 8),
        in_specs=[
            pl.BlockSpec((8, c, w), lambda j, i: (i, 0, j)),
            pl.BlockSpec((8, w), lambda j, i: (0, j)),
        ],
        out_specs=pl.BlockSpec((8, c, w), lambda j, i: (i, 0, j)),
        out_shape=jax.ShapeDtypeStruct((b, c, n), x.dtype),
        compiler_params=pltpu.CompilerParams(
            dimension_semantics=("arbitrary", "arbitrary"),
        ),
    )(xt, mask_row)

    return jnp.transpose(out_t, (0, 2, 1)), keep_mask
